# Initial kernel scaffold; baseline (speedup 1.0000x reference)
import jax
import jax.numpy as jnp
from jax.experimental import pallas as pl


def kernel(x, edge_index, batch, W1, a_src1, a_dst1, b1, W2, a_src2, a_dst2, b2, Wfc, bfc):
    raise NotImplementedError("write your pallas kernel here")


# speedup vs baseline: 18.2765x; 18.2765x over previous
import functools

import jax
import jax.numpy as jnp
from jax import lax
from jax.experimental import pallas as pl
from jax.experimental.pallas import tpu as pltpu
from jax.experimental.pallas import tpu_sc as plsc

N = 10000
E = 320000
D_IN = 128
D_H = 64
SEQ_OUT = 12
D_OUT = 14
NB = 32

DA = 72
NC = 2
NS = 16
NW = NC * NS
EPW = E // NW
CH = 80
NCHUNK = EPW // CH
RPT = 624
RTAIL = N - RPT * NS

BN = 1000
GRID = N // BN

_mesh = plsc.VectorSubcoreMesh(
    core_axis_name="c", subcore_axis_name="s", num_cores=NC, num_subcores=NS)


def _pre_body(f_ref, w_ref, ac_ref, haug_ref, init_ref, ad_ref):
    h = jnp.dot(f_ref[...], w_ref[...], preferred_element_type=jnp.float32)
    ha = jnp.dot(h, ac_ref[...], preferred_element_type=jnp.float32)
    t = ha[:, 0:1] + ha[:, 1:2]
    wself = jnp.exp(jnp.maximum(t, 0.2 * t))
    haug = jnp.concatenate(
        [h, jnp.ones((h.shape[0], 1), jnp.float32), ha[:, 0:1],
         jnp.zeros((h.shape[0], DA - D_H - 2), jnp.float32)], axis=1)
    haug_ref[...] = haug
    init_ref[0] = wself * haug
    init_ref[1] = jnp.zeros_like(haug)
    ad_ref[...] = ha[:, 1:2]


def _pre_call(f, W, ac):
    d_in = f.shape[1]
    return pl.pallas_call(
        _pre_body,
        grid=(GRID,),
        in_specs=[
            pl.BlockSpec((BN, d_in), lambda i: (i, 0)),
            pl.BlockSpec((d_in, D_H), lambda i: (0, 0)),
            pl.BlockSpec((D_H, 2), lambda i: (0, 0)),
        ],
        out_specs=[
            pl.BlockSpec((BN, DA), lambda i: (i, 0)),
            pl.BlockSpec((2, BN, DA), lambda i: (0, i, 0)),
            pl.BlockSpec((BN, 1), lambda i: (i, 0)),
        ],
        out_shape=[
            jax.ShapeDtypeStruct((N, DA), jnp.float32),
            jax.ShapeDtypeStruct((2, N, DA), jnp.float32),
            jax.ShapeDtypeStruct((N, 1), jnp.float32),
        ],
    )(f, W, ac)


def _combine_body(pair_ref, b_ref, out_ref):
    o = pair_ref[0] + pair_ref[1]
    s = o[:, D_H:D_H + 1]
    out_ref[...] = jnp.maximum(o[:, :D_H] / (s + 1e-16) + b_ref[...], 0.0)


def _combine_call(pair, b):
    return pl.pallas_call(
        _combine_body,
        grid=(GRID,),
        in_specs=[
            pl.BlockSpec((2, BN, DA), lambda i: (0, i, 0)),
            pl.BlockSpec((1, D_H), lambda i: (0, 0)),
        ],
        out_specs=pl.BlockSpec((BN, D_H), lambda i: (i, 0)),
        out_shape=jax.ShapeDtypeStruct((N, D_H), jnp.float32),
    )(pair, b.reshape(1, D_H))


def _pool_body(h_ref, batch_ref, wfc_ref, bfc_ref, out_ref, acc_ref):
    i = pl.program_id(0)
    h = h_ref[...]
    bt = batch_ref[0, 0, :]
    oh = (lax.broadcasted_iota(jnp.int32, (NB, BN), 0) == bt[None, :])
    oh = oh.astype(jnp.float32)
    ha = jnp.concatenate(
        [h, jnp.ones((BN, 1), jnp.float32), jnp.zeros((BN, 7), jnp.float32)],
        axis=1)
    contrib = jnp.dot(oh, ha, preferred_element_type=jnp.float32)

    @pl.when(i == 0)
    def _():
        acc_ref[...] = contrib

    @pl.when(i > 0)
    def _():
        acc_ref[...] += contrib

    @pl.when(i == pl.num_programs(0) - 1)
    def _():
        pa = acc_ref[...]
        pooled = pa[:, :D_H] / jnp.maximum(pa[:, D_H:D_H + 1], 1.0)
        out_ref[...] = (
            jnp.dot(pooled, wfc_ref[...], preferred_element_type=jnp.float32)
            + bfc_ref[...])


def _pool_call(h, batch3, Wfc, bfc):
    nfc = Wfc.shape[1]
    return pl.pallas_call(
        _pool_body,
        grid=(GRID,),
        in_specs=[
            pl.BlockSpec((BN, D_H), lambda i: (i, 0)),
            pl.BlockSpec((1, 1, BN), lambda i: (i, 0, 0)),
            pl.BlockSpec((D_H, nfc), lambda i: (0, 0)),
            pl.BlockSpec((1, nfc), lambda i: (0, 0)),
        ],
        out_specs=pl.BlockSpec((NB, nfc), lambda i: (0, 0)),
        out_shape=jax.ShapeDtypeStruct((NB, nfc), jnp.float32),
        scratch_shapes=[pltpu.VMEM((NB, D_H + 8), jnp.float32)],
    )(h, batch3, Wfc, bfc.reshape(1, nfc))


@functools.partial(
    pl.kernel,
    out_type=jax.ShapeDtypeStruct((2, N, DA), jnp.float32),
    mesh=_mesh,
    scratch_types=[
        pltpu.VMEM_SHARED((N, DA), jnp.float32),
        pltpu.VMEM((N,), jnp.float32),
        pltpu.VMEM((CH,), jnp.int32),
        pltpu.VMEM((CH,), jnp.int32),
        pltpu.VMEM((CH, DA), jnp.float32),
        pltpu.SemaphoreType.DMA,
    ],
    compiler_params=pltpu.CompilerParams(
        needs_layout_passes=False, use_tc_tiling_on_sc=False),
)
def _edge_kernel(haug_hbm, init_hbm, ad_hbm, src_hbm, dst_hbm, out_hbm,
                 acc_sh, ad_t, src_c, dst_c, rows, sem):
    c = lax.axis_index("c")
    s = lax.axis_index("s")
    wid = c * NS + s

    pltpu.sync_copy(ad_hbm, ad_t)
    r0 = s * RPT
    pltpu.sync_copy(init_hbm.at[c, pl.ds(r0, RPT)], acc_sh.at[pl.ds(r0, RPT)])

    @pl.when(s == 0)
    def _():
        pltpu.sync_copy(init_hbm.at[c, pl.ds(RPT * NS, RTAIL)],
                        acc_sh.at[pl.ds(RPT * NS, RTAIL)])

    plsc.subcore_barrier()

    ebase = wid * EPW
    c_as = jnp.full((16,), D_H + 1, jnp.int32)
    iota16 = jnp.arange(16, dtype=jnp.int32)

    def chunk_body(g, carry):
        cb = ebase + g * CH
        pltpu.sync_copy(src_hbm.at[pl.ds(cb, CH)], src_c)
        pltpu.sync_copy(dst_hbm.at[pl.ds(cb, CH)], dst_c)
        pltpu.async_copy(haug_hbm.at[src_c], rows, sem).wait()

        def group_body(j, carry2):
            sl = pl.ds(j * 16, 16)
            dv = dst_c[sl]
            rowi = iota16 + j * 16
            av = plsc.load_gather(rows, [rowi, c_as])
            bv = plsc.load_gather(ad_t, [dv])
            t = av + bv
            w = jnp.exp(jnp.maximum(t, 0.2 * t))

            def col_body(cc, carry3):
                cv = jnp.full((16,), 0, jnp.int32) + cc
                colv = plsc.load_gather(rows, [rowi, cv])
                plsc.store_scatter(rows, [rowi, cv], colv * w)
                return carry3

            return lax.fori_loop(0, D_H + 1, col_body, carry2)

        lax.fori_loop(0, CH // 16, group_body, 0)
        pltpu.sync_copy(rows, acc_sh.at[dst_c], add=True)
        return carry

    lax.fori_loop(0, NCHUNK, chunk_body, 0)

    plsc.subcore_barrier()
    pltpu.sync_copy(acc_sh.at[pl.ds(r0, RPT)], out_hbm.at[c, pl.ds(r0, RPT)])

    @pl.when(s == 0)
    def _():
        pltpu.sync_copy(acc_sh.at[pl.ds(RPT * NS, RTAIL)],
                        out_hbm.at[c, pl.ds(RPT * NS, RTAIL)])


def _gat_layer(f, edge_src, edge_dst, W, a_src, a_dst, b):
    ac = jnp.stack([a_src, a_dst], axis=1)
    haug, init_pair, ad = _pre_call(f, W, ac)
    pair = _edge_kernel(haug, init_pair, ad.reshape(N), edge_src, edge_dst)
    return _combine_call(pair, b)


def kernel(x, edge_index, batch, W1, a_src1, a_dst1, b1,
           W2, a_src2, a_dst2, b2, Wfc, bfc):
    src = edge_index[0]
    dst = edge_index[1]
    h = _gat_layer(x, src, dst, W1, a_src1, a_dst1, b1)
    h = _gat_layer(h, src, dst, W2, a_src2, a_dst2, b2)
    batch3 = batch.reshape(GRID, 1, BN)
    out = _pool_call(h, batch3, Wfc, bfc)
    return out.reshape(-1, SEQ_OUT, D_OUT)


# speedup vs baseline: 23.5441x; 1.2882x over previous
import functools

import jax
import jax.numpy as jnp
from jax import lax
from jax.experimental import pallas as pl
from jax.experimental.pallas import tpu as pltpu
from jax.experimental.pallas import tpu_sc as plsc

N = 10000
E = 320000
D_IN = 128
D_H = 64
SEQ_OUT = 12
D_OUT = 14
NB = 32

DA = 72
NC = 2
NS = 16
NW = NC * NS
EPW = E // NW
CH = 80
NCHUNK = EPW // CH
RPT = 624
RTAIL = N - RPT * NS

BN = 1000
GRID = N // BN

_mesh = plsc.VectorSubcoreMesh(
    core_axis_name="c", subcore_axis_name="s", num_cores=NC, num_subcores=NS)


def _pre_body(f_ref, w_ref, ac_ref, haug_ref, init_ref, ad_ref):
    h = jnp.dot(f_ref[...], w_ref[...], preferred_element_type=jnp.float32)
    ha = jnp.dot(h, ac_ref[...], preferred_element_type=jnp.float32)
    t = ha[:, 0:1] + ha[:, 1:2]
    wself = jnp.exp(jnp.maximum(t, 0.2 * t))
    haug = jnp.concatenate(
        [h, jnp.ones((h.shape[0], 1), jnp.float32), ha[:, 0:1],
         jnp.zeros((h.shape[0], DA - D_H - 2), jnp.float32)], axis=1)
    haug_ref[...] = haug
    init_ref[0] = wself * haug
    init_ref[1] = jnp.zeros_like(haug)
    ad_ref[...] = jnp.concatenate(
        [ha[:, 1:2], jnp.zeros((h.shape[0], 7), jnp.float32)], axis=1)


def _pre_call(f, W, ac):
    d_in = f.shape[1]
    return pl.pallas_call(
        _pre_body,
        grid=(GRID,),
        in_specs=[
            pl.BlockSpec((BN, d_in), lambda i: (i, 0)),
            pl.BlockSpec((d_in, D_H), lambda i: (0, 0)),
            pl.BlockSpec((D_H, 2), lambda i: (0, 0)),
        ],
        out_specs=[
            pl.BlockSpec((BN, DA), lambda i: (i, 0)),
            pl.BlockSpec((2, BN, DA), lambda i: (0, i, 0)),
            pl.BlockSpec((BN, 8), lambda i: (i, 0)),
        ],
        out_shape=[
            jax.ShapeDtypeStruct((N, DA), jnp.float32),
            jax.ShapeDtypeStruct((2, N, DA), jnp.float32),
            jax.ShapeDtypeStruct((N, 8), jnp.float32),
        ],
    )(f, W, ac)


def _combine_body(pair_ref, b_ref, out_ref):
    o = pair_ref[0] + pair_ref[1]
    s = o[:, D_H:D_H + 1]
    out_ref[...] = jnp.maximum(o[:, :D_H] / (s + 1e-16) + b_ref[...], 0.0)


def _combine_call(pair, b):
    return pl.pallas_call(
        _combine_body,
        grid=(GRID,),
        in_specs=[
            pl.BlockSpec((2, BN, DA), lambda i: (0, i, 0)),
            pl.BlockSpec((1, D_H), lambda i: (0, 0)),
        ],
        out_specs=pl.BlockSpec((BN, D_H), lambda i: (i, 0)),
        out_shape=jax.ShapeDtypeStruct((N, D_H), jnp.float32),
    )(pair, b.reshape(1, D_H))


def _pool_body(h_ref, batch_ref, wfc_ref, bfc_ref, out_ref, acc_ref):
    i = pl.program_id(0)
    h = h_ref[...]
    bt = batch_ref[0, 0, :]
    oh = (lax.broadcasted_iota(jnp.int32, (NB, BN), 0) == bt[None, :])
    oh = oh.astype(jnp.float32)
    ha = jnp.concatenate(
        [h, jnp.ones((BN, 1), jnp.float32), jnp.zeros((BN, 7), jnp.float32)],
        axis=1)
    contrib = jnp.dot(oh, ha, preferred_element_type=jnp.float32)

    @pl.when(i == 0)
    def _():
        acc_ref[...] = contrib

    @pl.when(i > 0)
    def _():
        acc_ref[...] += contrib

    @pl.when(i == pl.num_programs(0) - 1)
    def _():
        pa = acc_ref[...]
        pooled = pa[:, :D_H] / jnp.maximum(pa[:, D_H:D_H + 1], 1.0)
        out_ref[...] = (
            jnp.dot(pooled, wfc_ref[...], preferred_element_type=jnp.float32)
            + bfc_ref[...])


def _pool_call(h, batch3, Wfc, bfc):
    nfc = Wfc.shape[1]
    return pl.pallas_call(
        _pool_body,
        grid=(GRID,),
        in_specs=[
            pl.BlockSpec((BN, D_H), lambda i: (i, 0)),
            pl.BlockSpec((1, 1, BN), lambda i: (i, 0, 0)),
            pl.BlockSpec((D_H, nfc), lambda i: (0, 0)),
            pl.BlockSpec((1, nfc), lambda i: (0, 0)),
        ],
        out_specs=pl.BlockSpec((NB, nfc), lambda i: (0, 0)),
        out_shape=jax.ShapeDtypeStruct((NB, nfc), jnp.float32),
        scratch_shapes=[pltpu.VMEM((NB, D_H + 8), jnp.float32)],
    )(h, batch3, Wfc, bfc.reshape(1, nfc))


NBUF = 2
NPAIR = (NCHUNK + NBUF - 1) // NBUF


@functools.partial(
    pl.kernel,
    out_type=jax.ShapeDtypeStruct((2, N, DA), jnp.float32),
    mesh=_mesh,
    scratch_types=(
        [pltpu.VMEM_SHARED((N, DA), jnp.float32)]
        + [pltpu.VMEM((CH, DA), jnp.float32)] * NBUF
        + [pltpu.VMEM((CH, 8), jnp.float32)] * NBUF
        + [pltpu.VMEM((CH,), jnp.int32)] * NBUF
        + [pltpu.VMEM((CH,), jnp.int32)] * NBUF
        + [pltpu.SemaphoreType.DMA] * (2 * NBUF)
    ),
    compiler_params=pltpu.CompilerParams(
        needs_layout_passes=False, use_tc_tiling_on_sc=False),
)
def _edge_kernel(haug_hbm, init_hbm, ad_hbm, src_hbm, dst_hbm, out_hbm,
                 acc_sh, rows0, rows1, adr0, adr1, src0, src1, dst0, dst1,
                 gsem0, gsem1, ssem0, ssem1):
    c = lax.axis_index("c")
    s = lax.axis_index("s")
    wid = c * NS + s
    ROWS = (rows0, rows1)
    ADR = (adr0, adr1)
    SRC = (src0, src1)
    DST = (dst0, dst1)
    GSEM = (gsem0, gsem1)
    SSEM = (ssem0, ssem1)

    r0 = s * RPT
    pltpu.sync_copy(init_hbm.at[c, pl.ds(r0, RPT)], acc_sh.at[pl.ds(r0, RPT)])

    @pl.when(s == 0)
    def _():
        pltpu.sync_copy(init_hbm.at[c, pl.ds(RPT * NS, RTAIL)],
                        acc_sh.at[pl.ds(RPT * NS, RTAIL)])

    plsc.subcore_barrier()

    ebase = wid * EPW
    c_as = jnp.full((16,), D_H + 1, jnp.int32)
    iota16 = jnp.arange(16, dtype=jnp.int32)

    def issue(g, b):
        cb = ebase + g * CH
        pltpu.sync_copy(src_hbm.at[pl.ds(cb, CH)], SRC[b])
        pltpu.sync_copy(dst_hbm.at[pl.ds(cb, CH)], DST[b])
        pltpu.async_copy(haug_hbm.at[SRC[b]], ROWS[b], GSEM[b])
        pltpu.async_copy(ad_hbm.at[DST[b]], ADR[b], GSEM[b])

    def wait_gather(b):
        pltpu.make_async_copy(haug_hbm.at[SRC[b]], ROWS[b], GSEM[b]).wait()
        pltpu.make_async_copy(ad_hbm.at[DST[b]], ADR[b], GSEM[b]).wait()

    def drain_scatter(b):
        pltpu.make_async_copy(ROWS[b], acc_sh.at[DST[b]], SSEM[b]).wait()

    def compute(b):
        def group_body(j, carry2):
            rowi = iota16 + j * 16
            av = plsc.load_gather(ROWS[b], [rowi, c_as])
            bv = plsc.load_gather(ADR[b], [rowi, jnp.zeros((16,), jnp.int32)])
            t = av + bv
            w = jnp.exp(jnp.maximum(t, 0.2 * t))

            def col_body(cc, carry3):
                cv = jnp.full((16,), 0, jnp.int32) + cc
                colv = plsc.load_gather(ROWS[b], [rowi, cv])
                plsc.store_scatter(ROWS[b], [rowi, cv], colv * w)
                return carry3

            return lax.fori_loop(0, D_H + 1, col_body, carry2)

        lax.fori_loop(0, CH // 16, group_body, 0)
        pltpu.async_copy(ROWS[b], acc_sh.at[DST[b]], SSEM[b], add=True)

    for b in range(NBUF):
        issue(jnp.int32(b), b)

    def pair_body(i, carry):
        for b in range(NBUF):
            g = NBUF * i + b

            @pl.when(g < NCHUNK)
            def _():
                wait_gather(b)
                compute(b)

        for b in range(NBUF):
            g2 = NBUF * (i + 1) + b

            @pl.when(g2 < NCHUNK)
            def _():
                drain_scatter(b)
                issue(g2, b)

        return carry

    lax.fori_loop(0, NPAIR, pair_body, 0)

    for b in range(NBUF):
        drain_scatter(b)

    plsc.subcore_barrier()
    pltpu.sync_copy(acc_sh.at[pl.ds(r0, RPT)], out_hbm.at[c, pl.ds(r0, RPT)])

    @pl.when(s == 0)
    def _():
        pltpu.sync_copy(acc_sh.at[pl.ds(RPT * NS, RTAIL)],
                        out_hbm.at[c, pl.ds(RPT * NS, RTAIL)])


def _gat_layer(f, edge_src, edge_dst, W, a_src, a_dst, b):
    ac = jnp.stack([a_src, a_dst], axis=1)
    haug, init_pair, ad = _pre_call(f, W, ac)
    pair = _edge_kernel(haug, init_pair, ad, edge_src, edge_dst)
    return _combine_call(pair, b)


def kernel(x, edge_index, batch, W1, a_src1, a_dst1, b1,
           W2, a_src2, a_dst2, b2, Wfc, bfc):
    src = edge_index[0]
    dst = edge_index[1]
    h = _gat_layer(x, src, dst, W1, a_src1, a_dst1, b1)
    h = _gat_layer(h, src, dst, W2, a_src2, a_dst2, b2)
    batch3 = batch.reshape(GRID, 1, BN)
    out = _pool_call(h, batch3, Wfc, bfc)
    return out.reshape(-1, SEQ_OUT, D_OUT)


# speedup vs baseline: 23.9079x; 1.0155x over previous
import functools

import jax
import jax.numpy as jnp
from jax import lax
from jax.experimental import pallas as pl
from jax.experimental.pallas import tpu as pltpu
from jax.experimental.pallas import tpu_sc as plsc

N = 10000
E = 320000
D_IN = 128
D_H = 64
SEQ_OUT = 12
D_OUT = 14
NB = 32

DA = 72
NC = 2
NS = 16
NW = NC * NS
EPW = E // NW
CH = 80
NCHUNK = EPW // CH
RPT = 624
RTAIL = N - RPT * NS

BN = 1000
GRID = N // BN

_mesh = plsc.VectorSubcoreMesh(
    core_axis_name="c", subcore_axis_name="s", num_cores=NC, num_subcores=NS)


def _pre_body(f_ref, w_ref, ac_ref, haug_ref, init_ref, ad_ref):
    h = jnp.dot(f_ref[...], w_ref[...], preferred_element_type=jnp.float32)
    ha = jnp.dot(h, ac_ref[...], preferred_element_type=jnp.float32)
    t = ha[:, 0:1] + ha[:, 1:2]
    wself = jnp.exp(jnp.maximum(t, 0.2 * t))
    haug = jnp.concatenate(
        [h, jnp.ones((h.shape[0], 1), jnp.float32), ha[:, 0:1],
         jnp.zeros((h.shape[0], DA - D_H - 2), jnp.float32)], axis=1)
    haug_ref[...] = haug
    init_ref[0] = wself * haug
    init_ref[1] = jnp.zeros_like(haug)
    ad_ref[...] = jnp.concatenate(
        [ha[:, 1:2], jnp.zeros((h.shape[0], 7), jnp.float32)], axis=1)


def _pre_call(f, W, ac):
    d_in = f.shape[1]
    return pl.pallas_call(
        _pre_body,
        grid=(GRID,),
        in_specs=[
            pl.BlockSpec((BN, d_in), lambda i: (i, 0)),
            pl.BlockSpec((d_in, D_H), lambda i: (0, 0)),
            pl.BlockSpec((D_H, 2), lambda i: (0, 0)),
        ],
        out_specs=[
            pl.BlockSpec((BN, DA), lambda i: (i, 0)),
            pl.BlockSpec((2, BN, DA), lambda i: (0, i, 0)),
            pl.BlockSpec((BN, 8), lambda i: (i, 0)),
        ],
        out_shape=[
            jax.ShapeDtypeStruct((N, DA), jnp.float32),
            jax.ShapeDtypeStruct((2, N, DA), jnp.float32),
            jax.ShapeDtypeStruct((N, 8), jnp.float32),
        ],
    )(f, W, ac)


def _combine_body(pair_ref, b_ref, out_ref):
    o = pair_ref[0] + pair_ref[1]
    s = o[:, D_H:D_H + 1]
    out_ref[...] = jnp.maximum(o[:, :D_H] / (s + 1e-16) + b_ref[...], 0.0)


def _combine_call(pair, b):
    return pl.pallas_call(
        _combine_body,
        grid=(GRID,),
        in_specs=[
            pl.BlockSpec((2, BN, DA), lambda i: (0, i, 0)),
            pl.BlockSpec((1, D_H), lambda i: (0, 0)),
        ],
        out_specs=pl.BlockSpec((BN, D_H), lambda i: (i, 0)),
        out_shape=jax.ShapeDtypeStruct((N, D_H), jnp.float32),
    )(pair, b.reshape(1, D_H))


def _pool_body(h_ref, batch_ref, wfc_ref, bfc_ref, out_ref, acc_ref):
    i = pl.program_id(0)
    h = h_ref[...]
    bt = batch_ref[0, 0, :]
    oh = (lax.broadcasted_iota(jnp.int32, (NB, BN), 0) == bt[None, :])
    oh = oh.astype(jnp.float32)
    ha = jnp.concatenate(
        [h, jnp.ones((BN, 1), jnp.float32), jnp.zeros((BN, 7), jnp.float32)],
        axis=1)
    contrib = jnp.dot(oh, ha, preferred_element_type=jnp.float32)

    @pl.when(i == 0)
    def _():
        acc_ref[...] = contrib

    @pl.when(i > 0)
    def _():
        acc_ref[...] += contrib

    @pl.when(i == pl.num_programs(0) - 1)
    def _():
        pa = acc_ref[...]
        pooled = pa[:, :D_H] / jnp.maximum(pa[:, D_H:D_H + 1], 1.0)
        out_ref[...] = (
            jnp.dot(pooled, wfc_ref[...], preferred_element_type=jnp.float32)
            + bfc_ref[...])


def _pool_call(h, batch3, Wfc, bfc):
    nfc = Wfc.shape[1]
    return pl.pallas_call(
        _pool_body,
        grid=(GRID,),
        in_specs=[
            pl.BlockSpec((BN, D_H), lambda i: (i, 0)),
            pl.BlockSpec((1, 1, BN), lambda i: (i, 0, 0)),
            pl.BlockSpec((D_H, nfc), lambda i: (0, 0)),
            pl.BlockSpec((1, nfc), lambda i: (0, 0)),
        ],
        out_specs=pl.BlockSpec((NB, nfc), lambda i: (0, 0)),
        out_shape=jax.ShapeDtypeStruct((NB, nfc), jnp.float32),
        scratch_shapes=[pltpu.VMEM((NB, D_H + 8), jnp.float32)],
    )(h, batch3, Wfc, bfc.reshape(1, nfc))


NBUF = 3
NPAIR = (NCHUNK + NBUF - 1) // NBUF


@functools.partial(
    pl.kernel,
    out_type=jax.ShapeDtypeStruct((2, N, DA), jnp.float32),
    mesh=_mesh,
    scratch_types=(
        [pltpu.VMEM_SHARED((N, DA), jnp.float32)]
        + [pltpu.VMEM((CH, DA), jnp.float32)] * NBUF
        + [pltpu.VMEM((CH, 8), jnp.float32)] * NBUF
        + [pltpu.VMEM((CH,), jnp.int32)] * NBUF
        + [pltpu.VMEM((CH,), jnp.int32)] * NBUF
        + [pltpu.SemaphoreType.DMA] * (2 * NBUF)
    ),
    compiler_params=pltpu.CompilerParams(
        needs_layout_passes=False, use_tc_tiling_on_sc=False),
)
def _edge_kernel(haug_hbm, init_hbm, ad_hbm, src_hbm, dst_hbm, out_hbm,
                 acc_sh, rows0, rows1, rows2, adr0, adr1, adr2,
                 src0, src1, src2, dst0, dst1, dst2,
                 gsem0, gsem1, gsem2, ssem0, ssem1, ssem2):
    c = lax.axis_index("c")
    s = lax.axis_index("s")
    wid = c * NS + s
    ROWS = (rows0, rows1, rows2)
    ADR = (adr0, adr1, adr2)
    SRC = (src0, src1, src2)
    DST = (dst0, dst1, dst2)
    GSEM = (gsem0, gsem1, gsem2)
    SSEM = (ssem0, ssem1, ssem2)

    r0 = s * RPT
    pltpu.sync_copy(init_hbm.at[c, pl.ds(r0, RPT)], acc_sh.at[pl.ds(r0, RPT)])

    @pl.when(s == 0)
    def _():
        pltpu.sync_copy(init_hbm.at[c, pl.ds(RPT * NS, RTAIL)],
                        acc_sh.at[pl.ds(RPT * NS, RTAIL)])

    plsc.subcore_barrier()

    ebase = wid * EPW
    c_as = jnp.full((16,), D_H + 1, jnp.int32)
    iota16 = jnp.arange(16, dtype=jnp.int32)

    def issue(g, b):
        cb = ebase + g * CH
        pltpu.sync_copy(src_hbm.at[pl.ds(cb, CH)], SRC[b])
        pltpu.sync_copy(dst_hbm.at[pl.ds(cb, CH)], DST[b])
        pltpu.async_copy(haug_hbm.at[SRC[b]], ROWS[b], GSEM[b])
        pltpu.async_copy(ad_hbm.at[DST[b]], ADR[b], GSEM[b])

    def wait_gather(b):
        pltpu.make_async_copy(haug_hbm.at[SRC[b]], ROWS[b], GSEM[b]).wait()
        pltpu.make_async_copy(ad_hbm.at[DST[b]], ADR[b], GSEM[b]).wait()

    def drain_scatter(b):
        pltpu.make_async_copy(ROWS[b], acc_sh.at[DST[b]], SSEM[b]).wait()

    def compute(b):
        def group_body(j, carry2):
            rowi = iota16 + j * 16
            av = plsc.load_gather(ROWS[b], [rowi, c_as])
            bv = plsc.load_gather(ADR[b], [rowi, jnp.zeros((16,), jnp.int32)])
            t = av + bv
            w = jnp.exp(jnp.maximum(t, 0.2 * t))

            def col_body(cc, carry3):
                cv = jnp.full((16,), 0, jnp.int32) + cc
                colv = plsc.load_gather(ROWS[b], [rowi, cv])
                plsc.store_scatter(ROWS[b], [rowi, cv], colv * w)
                return carry3

            return lax.fori_loop(0, D_H + 1, col_body, carry2)

        lax.fori_loop(0, CH // 16, group_body, 0)
        pltpu.async_copy(ROWS[b], acc_sh.at[DST[b]], SSEM[b], add=True)

    for b in range(NBUF):
        issue(jnp.int32(b), b)

    def pair_body(i, carry):
        for b in range(NBUF):
            g = NBUF * i + b

            @pl.when(g < NCHUNK)
            def _():
                wait_gather(b)
                compute(b)

        for b in range(NBUF):
            g2 = NBUF * (i + 1) + b

            @pl.when(g2 < NCHUNK)
            def _():
                drain_scatter(b)
                issue(g2, b)

        return carry

    lax.fori_loop(0, NPAIR, pair_body, 0)

    for b in range(NBUF):
        drain_scatter(b)

    plsc.subcore_barrier()
    pltpu.sync_copy(acc_sh.at[pl.ds(r0, RPT)], out_hbm.at[c, pl.ds(r0, RPT)])

    @pl.when(s == 0)
    def _():
        pltpu.sync_copy(acc_sh.at[pl.ds(RPT * NS, RTAIL)],
                        out_hbm.at[c, pl.ds(RPT * NS, RTAIL)])


def _gat_layer(f, edge_src, edge_dst, W, a_src, a_dst, b):
    ac = jnp.stack([a_src, a_dst], axis=1)
    haug, init_pair, ad = _pre_call(f, W, ac)
    pair = _edge_kernel(haug, init_pair, ad, edge_src, edge_dst)
    return _combine_call(pair, b)


def kernel(x, edge_index, batch, W1, a_src1, a_dst1, b1,
           W2, a_src2, a_dst2, b2, Wfc, bfc):
    src = edge_index[0]
    dst = edge_index[1]
    h = _gat_layer(x, src, dst, W1, a_src1, a_dst1, b1)
    h = _gat_layer(h, src, dst, W2, a_src2, a_dst2, b2)
    batch3 = batch.reshape(GRID, 1, BN)
    out = _pool_call(h, batch3, Wfc, bfc)
    return out.reshape(-1, SEQ_OUT, D_OUT)


# speedup vs baseline: 24.0354x; 1.0053x over previous
import functools

import jax
import jax.numpy as jnp
from jax import lax
from jax.experimental import pallas as pl
from jax.experimental.pallas import tpu as pltpu
from jax.experimental.pallas import tpu_sc as plsc

N = 10000
E = 320000
D_IN = 128
D_H = 64
SEQ_OUT = 12
D_OUT = 14
NB = 32

DA = 72
NC = 2
NS = 16
NW = NC * NS
EPW = E // NW
CH = 80
NCHUNK = EPW // CH
RPT = 624
RTAIL = N - RPT * NS

BN = 1000
GRID = N // BN

_mesh = plsc.VectorSubcoreMesh(
    core_axis_name="c", subcore_axis_name="s", num_cores=NC, num_subcores=NS)


def _pre_body(f_ref, w_ref, ac_ref, haug_ref, init_ref, ad_ref):
    h = jnp.dot(f_ref[...], w_ref[...], preferred_element_type=jnp.float32)
    ha = jnp.dot(h, ac_ref[...], preferred_element_type=jnp.float32)
    t = ha[:, 0:1] + ha[:, 1:2]
    wself = jnp.exp(jnp.maximum(t, 0.2 * t))
    haug = jnp.concatenate(
        [h, jnp.ones((h.shape[0], 1), jnp.float32), ha[:, 0:1],
         jnp.zeros((h.shape[0], DA - D_H - 2), jnp.float32)], axis=1)
    haug_ref[...] = haug
    init_ref[0] = wself * haug
    init_ref[1] = jnp.zeros_like(haug)
    ad_ref[...] = jnp.concatenate(
        [ha[:, 1:2], jnp.zeros((h.shape[0], 7), jnp.float32)], axis=1)


def _pre_call(f, W, ac):
    d_in = f.shape[1]
    return pl.pallas_call(
        _pre_body,
        grid=(GRID,),
        in_specs=[
            pl.BlockSpec((BN, d_in), lambda i: (i, 0)),
            pl.BlockSpec((d_in, D_H), lambda i: (0, 0)),
            pl.BlockSpec((D_H, 2), lambda i: (0, 0)),
        ],
        out_specs=[
            pl.BlockSpec((BN, DA), lambda i: (i, 0)),
            pl.BlockSpec((2, BN, DA), lambda i: (0, i, 0)),
            pl.BlockSpec((BN, 8), lambda i: (i, 0)),
        ],
        out_shape=[
            jax.ShapeDtypeStruct((N, DA), jnp.float32),
            jax.ShapeDtypeStruct((2, N, DA), jnp.float32),
            jax.ShapeDtypeStruct((N, 8), jnp.float32),
        ],
    )(f, W, ac)


def _combine_body(pair_ref, b_ref, out_ref):
    o = pair_ref[0] + pair_ref[1]
    s = o[:, D_H:D_H + 1]
    out_ref[...] = jnp.maximum(o[:, :D_H] / (s + 1e-16) + b_ref[...], 0.0)


def _combine_call(pair, b):
    return pl.pallas_call(
        _combine_body,
        grid=(GRID,),
        in_specs=[
            pl.BlockSpec((2, BN, DA), lambda i: (0, i, 0)),
            pl.BlockSpec((1, D_H), lambda i: (0, 0)),
        ],
        out_specs=pl.BlockSpec((BN, D_H), lambda i: (i, 0)),
        out_shape=jax.ShapeDtypeStruct((N, D_H), jnp.float32),
    )(pair, b.reshape(1, D_H))


def _pool_body(h_ref, batch_ref, wfc_ref, bfc_ref, out_ref, acc_ref):
    i = pl.program_id(0)
    h = h_ref[...]
    bt = batch_ref[0, 0, :]
    oh = (lax.broadcasted_iota(jnp.int32, (NB, BN), 0) == bt[None, :])
    oh = oh.astype(jnp.float32)
    ha = jnp.concatenate(
        [h, jnp.ones((BN, 1), jnp.float32), jnp.zeros((BN, 7), jnp.float32)],
        axis=1)
    contrib = jnp.dot(oh, ha, preferred_element_type=jnp.float32)

    @pl.when(i == 0)
    def _():
        acc_ref[...] = contrib

    @pl.when(i > 0)
    def _():
        acc_ref[...] += contrib

    @pl.when(i == pl.num_programs(0) - 1)
    def _():
        pa = acc_ref[...]
        pooled = pa[:, :D_H] / jnp.maximum(pa[:, D_H:D_H + 1], 1.0)
        out_ref[...] = (
            jnp.dot(pooled, wfc_ref[...], preferred_element_type=jnp.float32)
            + bfc_ref[...])


def _pool_call(h, batch3, Wfc, bfc):
    nfc = Wfc.shape[1]
    return pl.pallas_call(
        _pool_body,
        grid=(GRID,),
        in_specs=[
            pl.BlockSpec((BN, D_H), lambda i: (i, 0)),
            pl.BlockSpec((1, 1, BN), lambda i: (i, 0, 0)),
            pl.BlockSpec((D_H, nfc), lambda i: (0, 0)),
            pl.BlockSpec((1, nfc), lambda i: (0, 0)),
        ],
        out_specs=pl.BlockSpec((NB, nfc), lambda i: (0, 0)),
        out_shape=jax.ShapeDtypeStruct((NB, nfc), jnp.float32),
        scratch_shapes=[pltpu.VMEM((NB, D_H + 8), jnp.float32)],
    )(h, batch3, Wfc, bfc.reshape(1, nfc))


NBUF = 3
NPAIR = (NCHUNK + NBUF - 1) // NBUF


@functools.partial(
    pl.kernel,
    out_type=jax.ShapeDtypeStruct((2, N, DA), jnp.float32),
    mesh=_mesh,
    scratch_types=(
        [pltpu.VMEM_SHARED((N, DA), jnp.float32)]
        + [pltpu.VMEM((CH, DA), jnp.float32)] * NBUF
        + [pltpu.VMEM((CH, 8), jnp.float32)] * NBUF
        + [pltpu.VMEM((CH,), jnp.int32)] * NBUF
        + [pltpu.VMEM((CH,), jnp.int32)] * NBUF
        + [pltpu.SemaphoreType.DMA] * (2 * NBUF)
    ),
    compiler_params=pltpu.CompilerParams(
        needs_layout_passes=False, use_tc_tiling_on_sc=False),
)
def _edge_kernel(haug_hbm, init_hbm, ad_hbm, src_hbm, dst_hbm, out_hbm,
                 acc_sh, rows0, rows1, rows2, adr0, adr1, adr2,
                 src0, src1, src2, dst0, dst1, dst2,
                 gsem0, gsem1, gsem2, ssem0, ssem1, ssem2):
    c = lax.axis_index("c")
    s = lax.axis_index("s")
    wid = c * NS + s
    ROWS = (rows0, rows1, rows2)
    ADR = (adr0, adr1, adr2)
    SRC = (src0, src1, src2)
    DST = (dst0, dst1, dst2)
    GSEM = (gsem0, gsem1, gsem2)
    SSEM = (ssem0, ssem1, ssem2)

    r0 = s * RPT
    pltpu.sync_copy(init_hbm.at[c, pl.ds(r0, RPT)], acc_sh.at[pl.ds(r0, RPT)])

    @pl.when(s == 0)
    def _():
        pltpu.sync_copy(init_hbm.at[c, pl.ds(RPT * NS, RTAIL)],
                        acc_sh.at[pl.ds(RPT * NS, RTAIL)])

    plsc.subcore_barrier()

    ebase = wid * EPW
    c_as = jnp.full((16,), D_H + 1, jnp.int32)
    iota16 = jnp.arange(16, dtype=jnp.int32)

    def issue(g, b):
        cb = ebase + g * CH
        pltpu.sync_copy(src_hbm.at[pl.ds(cb, CH)], SRC[b])
        pltpu.sync_copy(dst_hbm.at[pl.ds(cb, CH)], DST[b])
        pltpu.async_copy(haug_hbm.at[SRC[b]], ROWS[b], GSEM[b])
        pltpu.async_copy(ad_hbm.at[DST[b]], ADR[b], GSEM[b])

    def wait_gather(b):
        pltpu.make_async_copy(haug_hbm.at[SRC[b]], ROWS[b], GSEM[b]).wait()
        pltpu.make_async_copy(ad_hbm.at[DST[b]], ADR[b], GSEM[b]).wait()

    def drain_scatter(b):
        pltpu.make_async_copy(ROWS[b], acc_sh.at[DST[b]], SSEM[b]).wait()

    def compute(b):
        def group_body(j, carry2):
            rowi = iota16 + j * 16
            av = plsc.load_gather(ROWS[b], [rowi, c_as])
            bv = plsc.load_gather(ADR[b], [rowi, jnp.zeros((16,), jnp.int32)])
            t = av + bv
            w = jnp.exp(jnp.maximum(t, 0.2 * t))

            for cc in range(D_H + 1):
                cv = jnp.full((16,), cc, jnp.int32)
                colv = plsc.load_gather(ROWS[b], [rowi, cv])
                plsc.store_scatter(ROWS[b], [rowi, cv], colv * w)
            return carry2

        lax.fori_loop(0, CH // 16, group_body, 0)
        pltpu.async_copy(ROWS[b], acc_sh.at[DST[b]], SSEM[b], add=True)

    for b in range(NBUF):
        issue(jnp.int32(b), b)

    def pair_body(i, carry):
        for b in range(NBUF):
            g = NBUF * i + b

            @pl.when(g < NCHUNK)
            def _():
                wait_gather(b)
                compute(b)

        for b in range(NBUF):
            g2 = NBUF * (i + 1) + b

            @pl.when(g2 < NCHUNK)
            def _():
                drain_scatter(b)
                issue(g2, b)

        return carry

    lax.fori_loop(0, NPAIR, pair_body, 0)

    for b in range(NBUF):
        drain_scatter(b)

    plsc.subcore_barrier()
    pltpu.sync_copy(acc_sh.at[pl.ds(r0, RPT)], out_hbm.at[c, pl.ds(r0, RPT)])

    @pl.when(s == 0)
    def _():
        pltpu.sync_copy(acc_sh.at[pl.ds(RPT * NS, RTAIL)],
                        out_hbm.at[c, pl.ds(RPT * NS, RTAIL)])


def _gat_layer(f, edge_src, edge_dst, W, a_src, a_dst, b):
    ac = jnp.stack([a_src, a_dst], axis=1)
    haug, init_pair, ad = _pre_call(f, W, ac)
    pair = _edge_kernel(haug, init_pair, ad, edge_src, edge_dst)
    return _combine_call(pair, b)


def kernel(x, edge_index, batch, W1, a_src1, a_dst1, b1,
           W2, a_src2, a_dst2, b2, Wfc, bfc):
    src = edge_index[0]
    dst = edge_index[1]
    h = _gat_layer(x, src, dst, W1, a_src1, a_dst1, b1)
    h = _gat_layer(h, src, dst, W2, a_src2, a_dst2, b2)
    batch3 = batch.reshape(GRID, 1, BN)
    out = _pool_call(h, batch3, Wfc, bfc)
    return out.reshape(-1, SEQ_OUT, D_OUT)
